# baseline (speedup 1.0000x reference)
import jax
import jax.numpy as jnp
from jax import lax
from jax.experimental import pallas as pl
from jax.experimental.pallas import tpu as pltpu

_EPS = 1e-5
_PAD = 128


def _conv_stats_kernel(h_w, x_ref, w_ref, b_ref, conv_ref, sum_ref, ssq_ref):
    H, W = h_w
    x = x_ref[0].astype(jnp.bfloat16)
    C, HW = x.shape

    z = jnp.zeros((C, _PAD), jnp.bfloat16)
    xp = jnp.concatenate([z, x, z], axis=1)

    lane = lax.broadcasted_iota(jnp.int32, (1, HW), 1)
    wid = lane % W
    w_ok = [wid >= 1, None, wid <= W - 2]

    acc = None
    for ky in range(3):
        for kx in range(3):
            t = 3 * ky + kx
            st = _PAD + (ky - 1) * W + (kx - 1)
            col = xp[:, st:st + HW]
            if w_ok[kx] is not None:
                col = jnp.where(w_ok[kx], col, jnp.bfloat16(0))
            d = jnp.dot(w_ref[:, t * C:(t + 1) * C], col,
                        preferred_element_type=jnp.float32)
            acc = d if acc is None else acc + d
    acc = acc + b_ref[...]

    conv_ref[0] = acc.astype(conv_ref.dtype)
    sum_ref[0] = jnp.sum(acc, axis=1, keepdims=True)
    ssq_ref[0] = jnp.sum(acc * acc, axis=1, keepdims=True)


def _bn_relu_kernel(conv_ref, scale_ref, shift_ref, o_ref):
    y = conv_ref[0].astype(jnp.float32) * scale_ref[...] + shift_ref[...]
    o_ref[0] = jnp.maximum(y, 0.0)


def kernel(x_nchw, w_oihw, bias, gamma, beta):
    N, C, H, W = x_nchw.shape
    HW = H * W
    x = x_nchw.reshape(N, C, HW)

    w_lhs = jnp.transpose(w_oihw, (0, 2, 3, 1)).reshape(C, 9 * C)
    w_lhs = w_lhs.astype(jnp.bfloat16)
    b_col = bias.reshape(C, 1).astype(jnp.float32)

    flops = 2 * N * HW * (9 * C) * C
    bytes_accessed = (N * C * HW) * (4 + 2) + (9 * C * C) * 2 + 2 * N * C * 4

    conv, psum, pssq = pl.pallas_call(
        lambda *refs: _conv_stats_kernel((H, W), *refs),
        out_shape=(
            jax.ShapeDtypeStruct((N, C, HW), jnp.bfloat16),
            jax.ShapeDtypeStruct((N, C, 1), jnp.float32),
            jax.ShapeDtypeStruct((N, C, 1), jnp.float32),
        ),
        grid=(N,),
        in_specs=[
            pl.BlockSpec((1, C, HW), lambda n: (n, 0, 0)),
            pl.BlockSpec((C, 9 * C), lambda n: (0, 0)),
            pl.BlockSpec((C, 1), lambda n: (0, 0)),
        ],
        out_specs=(
            pl.BlockSpec((1, C, HW), lambda n: (n, 0, 0)),
            pl.BlockSpec((1, C, 1), lambda n: (n, 0, 0)),
            pl.BlockSpec((1, C, 1), lambda n: (n, 0, 0)),
        ),
        compiler_params=pltpu.CompilerParams(dimension_semantics=("parallel",)),
        cost_estimate=pl.CostEstimate(
            flops=flops, transcendentals=0, bytes_accessed=bytes_accessed),
    )(x, w_lhs, b_col)

    m_total = float(N * HW)
    ch_sum = jnp.sum(psum, axis=0)
    ch_ssq = jnp.sum(pssq, axis=0)
    mean = ch_sum / m_total
    var = jnp.maximum(ch_ssq / m_total - mean * mean, 0.0)
    inv = lax.rsqrt(var + _EPS)
    scale = gamma.reshape(C, 1).astype(jnp.float32) * inv
    shift = beta.reshape(C, 1).astype(jnp.float32) - mean * scale

    y = pl.pallas_call(
        _bn_relu_kernel,
        out_shape=jax.ShapeDtypeStruct((N, C, HW), jnp.float32),
        grid=(N,),
        in_specs=[
            pl.BlockSpec((1, C, HW), lambda n: (n, 0, 0)),
            pl.BlockSpec((C, 1), lambda n: (0, 0)),
            pl.BlockSpec((C, 1), lambda n: (0, 0)),
        ],
        out_specs=pl.BlockSpec((1, C, HW), lambda n: (n, 0, 0)),
        compiler_params=pltpu.CompilerParams(dimension_semantics=("parallel",)),
        cost_estimate=pl.CostEstimate(
            flops=5 * N * C * HW, transcendentals=C,
            bytes_accessed=(N * C * HW) * (2 + 4)),
    )(conv, scale, shift)

    return y.reshape(N, C, H, W)


# speedup vs baseline: 1.0489x; 1.0489x over previous
import jax
import jax.numpy as jnp
from jax import lax
from jax.experimental import pallas as pl
from jax.experimental.pallas import tpu as pltpu

_EPS = 1e-5
_PAD = 128


def _fused_kernel(dims, x_ref, w_ref, b_ref, g_ref, be_ref, o_ref,
                  conv_scr, stat_scr, ss_scr):
    H, W, N = dims
    ph = pl.program_id(0)
    n = pl.program_id(1)

    @pl.when(ph == 0)
    def _conv_phase():
        x = x_ref[0].astype(jnp.bfloat16)
        C, HW = x.shape
        z = jnp.zeros((C, _PAD), jnp.bfloat16)
        xp = jnp.concatenate([z, x, z], axis=1)

        lane = lax.broadcasted_iota(jnp.int32, (1, HW), 1)
        wid = lane % W
        w_ok = [wid >= 1, None, wid <= W - 2]

        acc = None
        for ky in range(3):
            for kx in range(3):
                t = 3 * ky + kx
                st = _PAD + (ky - 1) * W + (kx - 1)
                col = xp[:, st:st + HW]
                if w_ok[kx] is not None:
                    col = jnp.where(w_ok[kx], col, jnp.bfloat16(0))
                d = jnp.dot(w_ref[:, t * C:(t + 1) * C], col,
                            preferred_element_type=jnp.float32)
                acc = d if acc is None else acc + d
        acc = acc + b_ref[...]

        conv_scr[n] = acc.astype(conv_scr.dtype)

        @pl.when(n == 0)
        def _init_stats():
            stat_scr[0] = jnp.zeros_like(stat_scr[0])
            stat_scr[1] = jnp.zeros_like(stat_scr[1])

        stat_scr[0] += jnp.sum(acc, axis=1, keepdims=True)
        stat_scr[1] += jnp.sum(acc * acc, axis=1, keepdims=True)

    @pl.when(ph == 1)
    def _bn_phase():
        @pl.when(n == 0)
        def _finalize():
            m_total = float(N * H * W)
            mean = stat_scr[0] / m_total
            var = jnp.maximum(stat_scr[1] / m_total - mean * mean, 0.0)
            inv = lax.rsqrt(var + _EPS)
            scale = g_ref[...] * inv
            ss_scr[0] = scale
            ss_scr[1] = be_ref[...] - mean * scale

        y = conv_scr[n].astype(jnp.float32) * ss_scr[0] + ss_scr[1]
        o_ref[0] = jnp.maximum(y, 0.0)


def kernel(x_nchw, w_oihw, bias, gamma, beta):
    N, C, H, W = x_nchw.shape
    HW = H * W
    x = x_nchw.reshape(N, C, HW)

    w_lhs = jnp.transpose(w_oihw, (0, 2, 3, 1)).reshape(C, 9 * C)
    w_lhs = w_lhs.astype(jnp.bfloat16)
    b_col = bias.reshape(C, 1).astype(jnp.float32)
    g_col = gamma.reshape(C, 1).astype(jnp.float32)
    be_col = beta.reshape(C, 1).astype(jnp.float32)

    flops = 2 * N * HW * (9 * C) * C + 5 * N * C * HW
    bytes_accessed = (N * C * HW) * (4 + 4) + (9 * C * C) * 2

    y = pl.pallas_call(
        lambda *refs: _fused_kernel((H, W, N), *refs),
        out_shape=jax.ShapeDtypeStruct((N, C, HW), jnp.float32),
        grid=(2, N),
        in_specs=[
            pl.BlockSpec((1, C, HW),
                         lambda ph, n: (n * (1 - ph) + (N - 1) * ph, 0, 0)),
            pl.BlockSpec((C, 9 * C), lambda ph, n: (0, 0)),
            pl.BlockSpec((C, 1), lambda ph, n: (0, 0)),
            pl.BlockSpec((C, 1), lambda ph, n: (0, 0)),
            pl.BlockSpec((C, 1), lambda ph, n: (0, 0)),
        ],
        out_specs=pl.BlockSpec((1, C, HW), lambda ph, n: (n * ph, 0, 0)),
        scratch_shapes=[
            pltpu.VMEM((N, C, HW), jnp.bfloat16),
            pltpu.VMEM((2, C, 1), jnp.float32),
            pltpu.VMEM((2, C, 1), jnp.float32),
        ],
        compiler_params=pltpu.CompilerParams(
            dimension_semantics=("arbitrary", "arbitrary")),
        cost_estimate=pl.CostEstimate(
            flops=flops, transcendentals=C, bytes_accessed=bytes_accessed),
    )(x, w_lhs, b_col, g_col, be_col)

    return y.reshape(N, C, H, W)


# speedup vs baseline: 1.2003x; 1.1443x over previous
import jax
import jax.numpy as jnp
from jax import lax
from jax.experimental import pallas as pl
from jax.experimental.pallas import tpu as pltpu

_EPS = 1e-5


def _roll_lanes(x, k, size):
    k %= size
    if k == 0:
        return x
    return jnp.concatenate([x[:, k:], x[:, :k]], axis=1)


def _conv_stats_kernel(h_w, x_ref, w_ref, b_ref, conv_ref, sum_ref, ssq_ref):
    H, W = h_w
    x = x_ref[0].astype(jnp.bfloat16)
    C, HW = x.shape

    lane = lax.broadcasted_iota(jnp.int32, (1, HW), 1)
    wid = lane % W
    hid = lane // W
    w_ok = [wid >= 1, None, wid <= W - 2]
    h_ok = [hid >= 1, None, hid <= H - 2]

    cols = []
    for ky in range(3):
        for kx in range(3):
            off = (ky - 1) * W + (kx - 1)
            col = _roll_lanes(x, off, HW)
            m = h_ok[ky] if w_ok[kx] is None else (
                w_ok[kx] if h_ok[ky] is None else jnp.logical_and(h_ok[ky], w_ok[kx]))
            if m is not None:
                col = jnp.where(m, col, jnp.bfloat16(0))
            cols.append(col)
    patches = jnp.concatenate(cols, axis=0)

    acc = jnp.dot(w_ref[...], patches, preferred_element_type=jnp.float32)
    acc = acc + b_ref[...]

    conv_ref[0] = acc.astype(conv_ref.dtype)
    sum_ref[0] = jnp.sum(acc, axis=1, keepdims=True)
    ssq_ref[0] = jnp.sum(acc * acc, axis=1, keepdims=True)


def _bn_relu_kernel(conv_ref, scale_ref, shift_ref, o_ref):
    y = conv_ref[0].astype(jnp.float32) * scale_ref[...] + shift_ref[...]
    o_ref[0] = jnp.maximum(y, 0.0)


def kernel(x_nchw, w_oihw, bias, gamma, beta):
    N, C, H, W = x_nchw.shape
    HW = H * W
    x = x_nchw.reshape(N, C, HW)

    w_lhs = jnp.transpose(w_oihw, (0, 2, 3, 1)).reshape(C, 9 * C)
    w_lhs = w_lhs.astype(jnp.bfloat16)
    b_col = bias.reshape(C, 1).astype(jnp.float32)

    flops = 2 * N * HW * (9 * C) * C
    bytes_accessed = (N * C * HW) * (4 + 2) + (9 * C * C) * 2 + 2 * N * C * 4

    conv, psum, pssq = pl.pallas_call(
        lambda *refs: _conv_stats_kernel((H, W), *refs),
        out_shape=(
            jax.ShapeDtypeStruct((N, C, HW), jnp.bfloat16),
            jax.ShapeDtypeStruct((N, C, 1), jnp.float32),
            jax.ShapeDtypeStruct((N, C, 1), jnp.float32),
        ),
        grid=(N,),
        in_specs=[
            pl.BlockSpec((1, C, HW), lambda n: (n, 0, 0)),
            pl.BlockSpec((C, 9 * C), lambda n: (0, 0)),
            pl.BlockSpec((C, 1), lambda n: (0, 0)),
        ],
        out_specs=(
            pl.BlockSpec((1, C, HW), lambda n: (n, 0, 0)),
            pl.BlockSpec((1, C, 1), lambda n: (n, 0, 0)),
            pl.BlockSpec((1, C, 1), lambda n: (n, 0, 0)),
        ),
        compiler_params=pltpu.CompilerParams(dimension_semantics=("parallel",)),
        cost_estimate=pl.CostEstimate(
            flops=flops, transcendentals=0, bytes_accessed=bytes_accessed),
    )(x, w_lhs, b_col)

    m_total = float(N * HW)
    ch_sum = jnp.sum(psum, axis=0)
    ch_ssq = jnp.sum(pssq, axis=0)
    mean = ch_sum / m_total
    var = jnp.maximum(ch_ssq / m_total - mean * mean, 0.0)
    inv = lax.rsqrt(var + _EPS)
    scale = gamma.reshape(C, 1).astype(jnp.float32) * inv
    shift = beta.reshape(C, 1).astype(jnp.float32) - mean * scale

    y = pl.pallas_call(
        _bn_relu_kernel,
        out_shape=jax.ShapeDtypeStruct((N, C, HW), jnp.float32),
        grid=(N,),
        in_specs=[
            pl.BlockSpec((1, C, HW), lambda n: (n, 0, 0)),
            pl.BlockSpec((C, 1), lambda n: (0, 0)),
            pl.BlockSpec((C, 1), lambda n: (0, 0)),
        ],
        out_specs=pl.BlockSpec((1, C, HW), lambda n: (n, 0, 0)),
        compiler_params=pltpu.CompilerParams(dimension_semantics=("parallel",)),
    )(conv, scale, shift)

    return y.reshape(N, C, H, W)


# speedup vs baseline: 1.2657x; 1.0545x over previous
import jax
import jax.numpy as jnp
from jax import lax
from jax.experimental import pallas as pl
from jax.experimental.pallas import tpu as pltpu

_EPS = 1e-5


def _roll_lanes(x, k, size):
    k %= size
    if k == 0:
        return x
    return jnp.concatenate([x[:, k:], x[:, :k]], axis=1)


def _conv_stats_kernel(h_w, x_ref, w_ref, b_ref, conv_ref, sum_ref, ssq_ref):
    H, W = h_w
    B = x_ref.shape[0]
    C, HW = x_ref.shape[1], x_ref.shape[2]

    lane = lax.broadcasted_iota(jnp.int32, (1, HW), 1)
    wid = lane % W
    hid = lane // W
    w_ok = [wid >= 1, None, wid <= W - 2]
    h_ok = [hid >= 1, None, hid <= H - 2]

    for b in range(B):
        x = x_ref[b].astype(jnp.bfloat16)
        cols = []
        for ky in range(3):
            for kx in range(3):
                off = (ky - 1) * W + (kx - 1)
                col = _roll_lanes(x, off, HW)
                m = h_ok[ky] if w_ok[kx] is None else (
                    w_ok[kx] if h_ok[ky] is None else jnp.logical_and(h_ok[ky], w_ok[kx]))
                if m is not None:
                    col = jnp.where(m, col, jnp.bfloat16(0))
                cols.append(col)
        patches = jnp.concatenate(cols, axis=0)

        acc = jnp.dot(w_ref[...], patches, preferred_element_type=jnp.float32)
        acc = acc + b_ref[...]

        conv_ref[b] = acc.astype(conv_ref.dtype)
        sum_ref[b] = jnp.sum(acc, axis=1, keepdims=True)
        ssq_ref[b] = jnp.sum(acc * acc, axis=1, keepdims=True)


def _bn_relu_kernel(conv_ref, scale_ref, shift_ref, o_ref):
    y = conv_ref[...].astype(jnp.float32) * scale_ref[...] + shift_ref[...]
    o_ref[...] = jnp.maximum(y, 0.0)


def kernel(x_nchw, w_oihw, bias, gamma, beta):
    N, C, H, W = x_nchw.shape
    HW = H * W
    x = x_nchw.reshape(N, C, HW)

    w_lhs = jnp.transpose(w_oihw, (0, 2, 3, 1)).reshape(C, 9 * C)
    w_lhs = w_lhs.astype(jnp.bfloat16)
    b_col = bias.reshape(C, 1).astype(jnp.float32)

    flops = 2 * N * HW * (9 * C) * C
    bytes_accessed = (N * C * HW) * (4 + 2) + (9 * C * C) * 2 + 2 * N * C * 4

    conv, psum, pssq = pl.pallas_call(
        lambda *refs: _conv_stats_kernel((H, W), *refs),
        out_shape=(
            jax.ShapeDtypeStruct((N, C, HW), jnp.bfloat16),
            jax.ShapeDtypeStruct((N, C, 1), jnp.float32),
            jax.ShapeDtypeStruct((N, C, 1), jnp.float32),
        ),
        grid=(N // 2,),
        in_specs=[
            pl.BlockSpec((2, C, HW), lambda n: (n, 0, 0)),
            pl.BlockSpec((C, 9 * C), lambda n: (0, 0)),
            pl.BlockSpec((C, 1), lambda n: (0, 0)),
        ],
        out_specs=(
            pl.BlockSpec((2, C, HW), lambda n: (n, 0, 0)),
            pl.BlockSpec((2, C, 1), lambda n: (n, 0, 0)),
            pl.BlockSpec((2, C, 1), lambda n: (n, 0, 0)),
        ),
        compiler_params=pltpu.CompilerParams(dimension_semantics=("parallel",)),
        cost_estimate=pl.CostEstimate(
            flops=flops, transcendentals=0, bytes_accessed=bytes_accessed),
    )(x, w_lhs, b_col)

    m_total = float(N * HW)
    ch_sum = jnp.sum(psum, axis=0)
    ch_ssq = jnp.sum(pssq, axis=0)
    mean = ch_sum / m_total
    var = jnp.maximum(ch_ssq / m_total - mean * mean, 0.0)
    inv = lax.rsqrt(var + _EPS)
    scale = gamma.reshape(C, 1).astype(jnp.float32) * inv
    shift = beta.reshape(C, 1).astype(jnp.float32) - mean * scale

    y = pl.pallas_call(
        _bn_relu_kernel,
        out_shape=jax.ShapeDtypeStruct((N, C, HW), jnp.float32),
        grid=(N // 2,),
        in_specs=[
            pl.BlockSpec((2, C, HW), lambda n: (n, 0, 0)),
            pl.BlockSpec((C, 1), lambda n: (0, 0)),
            pl.BlockSpec((C, 1), lambda n: (0, 0)),
        ],
        out_specs=pl.BlockSpec((2, C, HW), lambda n: (n, 0, 0)),
        compiler_params=pltpu.CompilerParams(dimension_semantics=("parallel",)),
    )(conv, scale, shift)

    return y.reshape(N, C, H, W)


# speedup vs baseline: 1.2720x; 1.0050x over previous
import jax
import jax.numpy as jnp
from jax import lax
from jax.experimental import pallas as pl
from jax.experimental.pallas import tpu as pltpu

_EPS = 1e-5


def _roll_lanes(x, k, size):
    k %= size
    if k == 0:
        return x
    return jnp.concatenate([x[:, k:], x[:, :k]], axis=1)


def _conv_stats_kernel(h_w, x_ref, w_ref, b_ref, conv_ref, sum_ref, ssq_ref):
    H, W = h_w
    B = x_ref.shape[0]
    C, HW = x_ref.shape[1], x_ref.shape[2]

    lane = lax.broadcasted_iota(jnp.int32, (1, HW), 1)
    wid = lane % W
    hid = lane // W
    w_ok = [wid >= 1, None, wid <= W - 2]
    h_ok = [hid >= 1, None, hid <= H - 2]

    for b in range(B):
        x = x_ref[b].astype(jnp.bfloat16)
        cols = []
        for ky in range(3):
            for kx in range(3):
                off = (ky - 1) * W + (kx - 1)
                col = _roll_lanes(x, off, HW)
                m = h_ok[ky] if w_ok[kx] is None else (
                    w_ok[kx] if h_ok[ky] is None else jnp.logical_and(h_ok[ky], w_ok[kx]))
                if m is not None:
                    col = jnp.where(m, col, jnp.bfloat16(0))
                cols.append(col)
        patches = jnp.concatenate(cols, axis=0)

        acc = jnp.dot(w_ref[...], patches, preferred_element_type=jnp.float32)
        acc = acc + b_ref[...]

        conv_ref[b] = acc.astype(conv_ref.dtype)
        sum_ref[b] = jnp.sum(acc, axis=1, keepdims=True)
        ssq_ref[b] = jnp.sum(acc * acc, axis=1, keepdims=True)


def _bn_relu_kernel(conv_ref, scale_ref, shift_ref, o_ref):
    y = conv_ref[...].astype(jnp.float32) * scale_ref[...] + shift_ref[...]
    o_ref[...] = jnp.maximum(y, 0.0)


def kernel(x_nchw, w_oihw, bias, gamma, beta):
    N, C, H, W = x_nchw.shape
    HW = H * W
    x = x_nchw.reshape(N, C, HW)

    w_lhs = jnp.transpose(w_oihw, (0, 2, 3, 1)).reshape(C, 9 * C)
    w_lhs = w_lhs.astype(jnp.bfloat16)
    b_col = bias.reshape(C, 1).astype(jnp.float32)

    flops = 2 * N * HW * (9 * C) * C
    bytes_accessed = (N * C * HW) * (4 + 2) + (9 * C * C) * 2 + 2 * N * C * 4

    conv, psum, pssq = pl.pallas_call(
        lambda *refs: _conv_stats_kernel((H, W), *refs),
        out_shape=(
            jax.ShapeDtypeStruct((N, C, HW), jnp.bfloat16),
            jax.ShapeDtypeStruct((N, C, 1), jnp.float32),
            jax.ShapeDtypeStruct((N, C, 1), jnp.float32),
        ),
        grid=(N // 4,),
        in_specs=[
            pl.BlockSpec((4, C, HW), lambda n: (n, 0, 0)),
            pl.BlockSpec((C, 9 * C), lambda n: (0, 0)),
            pl.BlockSpec((C, 1), lambda n: (0, 0)),
        ],
        out_specs=(
            pl.BlockSpec((4, C, HW), lambda n: (n, 0, 0)),
            pl.BlockSpec((4, C, 1), lambda n: (n, 0, 0)),
            pl.BlockSpec((4, C, 1), lambda n: (n, 0, 0)),
        ),
        compiler_params=pltpu.CompilerParams(dimension_semantics=("parallel",)),
        cost_estimate=pl.CostEstimate(
            flops=flops, transcendentals=0, bytes_accessed=bytes_accessed),
    )(x, w_lhs, b_col)

    m_total = float(N * HW)
    ch_sum = jnp.sum(psum, axis=0)
    ch_ssq = jnp.sum(pssq, axis=0)
    mean = ch_sum / m_total
    var = jnp.maximum(ch_ssq / m_total - mean * mean, 0.0)
    inv = lax.rsqrt(var + _EPS)
    scale = gamma.reshape(C, 1).astype(jnp.float32) * inv
    shift = beta.reshape(C, 1).astype(jnp.float32) - mean * scale

    y = pl.pallas_call(
        _bn_relu_kernel,
        out_shape=jax.ShapeDtypeStruct((N, C, HW), jnp.float32),
        grid=(N // 4,),
        in_specs=[
            pl.BlockSpec((4, C, HW), lambda n: (n, 0, 0)),
            pl.BlockSpec((C, 1), lambda n: (0, 0)),
            pl.BlockSpec((C, 1), lambda n: (0, 0)),
        ],
        out_specs=pl.BlockSpec((4, C, HW), lambda n: (n, 0, 0)),
        compiler_params=pltpu.CompilerParams(dimension_semantics=("parallel",)),
    )(conv, scale, shift)

    return y.reshape(N, C, H, W)
